# baseline (speedup 1.0000x reference)
import functools

import numpy as np
import jax
import jax.numpy as jnp
from jax import lax
from jax.experimental import pallas as pl
from jax.experimental.pallas import tpu as pltpu

_PATCH = (1, 2, 4, 8, 16, 32)
_VOCAB = 8192
_C = 32
_HW = 32
_NPIX = _HW * _HW
_B = 16
_KIDX = (0, 0, 1, 2, 3, 3)
_BETA = 0.25
_NSCALES = 6


def _cubic_kernel(x):
    x = np.abs(x)
    out = np.where(x <= 1.0, ((1.5 * x - 2.5) * x) * x + 1.0, 0.0)
    out = np.where((x > 1.0) & (x < 2.0), ((-0.5 * x + 2.5) * x - 4.0) * x + 2.0, out)
    return out


def _resize_mat(n, m):
    sample_f = (np.arange(m, dtype=np.float64) + 0.5) * (n / m) - 0.5
    x = sample_f[None, :] - np.arange(n, dtype=np.float64)[:, None]
    w = _cubic_kernel(x)
    w = w / w.sum(axis=0, keepdims=True)
    return w.T


def _pool_mat(pn):
    f = _HW // pn
    a = np.zeros((pn, _HW), dtype=np.float64)
    for p in range(pn):
        a[p, p * f:(p + 1) * f] = 1.0 / f
    return a


_U_MATS = []
_P_MATS = []
for _pn in _PATCH[:-1]:
    _k = _resize_mat(_pn, _HW)
    _U_MATS.append(np.kron(_k, _k).astype(np.float32))
    _a = _pool_mat(_pn)
    _P_MATS.append(np.kron(_a, _a).astype(np.float32))

_CHUNK = 256


def _vq_body(x_ref, emb_ref, embt_ref, w_ref, b_ref,
             u0, u1, u2, u3, u4, p0, p1, p2, p3, p4,
             y_ref, loss_ref, pad_ref):
    u_refs = (u0, u1, u2, u3, u4)
    p_refs = (p0, p1, p2, p3, p4)

    emb = emb_ref[...]
    embt = embt_ref[...]
    e2 = jnp.sum(emb * emb, axis=1)[None, :]

    x_img = x_ref[0]
    resid = x_img
    recon = jnp.zeros((_NPIX, _C), jnp.float32)
    loss_acc = jnp.zeros((1, 1), jnp.float32)

    pad_ref[...] = jnp.zeros(pad_ref.shape, jnp.float32)

    ww = lax.broadcasted_iota(jnp.int32, (_NPIX, 1), 0) & (_HW - 1)
    mask_l = (ww >= 1).astype(jnp.float32)
    mask_r = (ww <= _HW - 2).astype(jnp.float32)

    for si, pn in enumerate(_PATCH):
        n = pn * pn
        if si < _NSCALES - 1:
            z = lax.dot_general(p_refs[si][...], resid,
                                (((1,), (0,)), ((), ())),
                                precision=lax.Precision.HIGHEST)
        else:
            z = resid
        x2 = jnp.sum(z * z, axis=1, keepdims=True)

        ch = min(n, _CHUNK)
        h_chunks = []
        for c0 in range(0, n, ch):
            zc = lax.slice(z, (c0, 0), (c0 + ch, _C))
            x2c = lax.slice(x2, (c0, 0), (c0 + ch, 1))
            xe = lax.dot_general(zc, embt, (((1,), (0,)), ((), ())),
                                 precision=None)
            d = (x2c + e2) - 2.0 * xe
            iv = lax.broadcasted_iota(jnp.int32, (ch, _VOCAB), 1)
            if si < _NSCALES - 1:
                m = jnp.min(d, axis=1, keepdims=True)
                idx = jnp.min(jnp.where(d == m, iv, _VOCAB), axis=1, keepdims=True)
            else:
                accv = acci = None
                for k in range(8):
                    dk = lax.slice(d, (0, k * 1024), (ch, (k + 1) * 1024))
                    ivk = lax.slice(iv, (0, k * 1024), (ch, (k + 1) * 1024))
                    mk = jnp.min(dk, axis=1, keepdims=True)
                    ik = jnp.min(jnp.where(dk == mk, ivk, _VOCAB),
                                 axis=1, keepdims=True)
                    if k == 0:
                        accv, acci = mk, ik
                    else:
                        take = mk <= accv
                        accv = jnp.where(take, mk, accv)
                        acci = jnp.where(take, ik, acci)
                    if k % 4 == 3 and k != 7:
                        accv = accv.astype(jnp.bfloat16).astype(jnp.float32)
                idx = acci
            oh = (iv == idx).astype(jnp.float32)
            h_chunks.append(lax.dot_general(oh, emb, (((1,), (0,)), ((), ())),
                                            precision=lax.Precision.HIGHEST))
        h_small = h_chunks[0] if len(h_chunks) == 1 else jnp.concatenate(h_chunks, 0)

        if si < _NSCALES - 1:
            h = lax.dot_general(u_refs[si][...], h_small,
                                (((1,), (0,)), ((), ())),
                                precision=lax.Precision.HIGHEST)
        else:
            h = h_small

        pad_ref[pl.ds(40, _NPIX), :] = h
        conv = None
        for ky in range(3):
            for kx in range(3):
                s = (ky - 1) * _HW + (kx - 1)
                hs = pad_ref[pl.ds(40 + s, _NPIX), :]
                if kx == 0:
                    hs = hs * mask_l
                elif kx == 2:
                    hs = hs * mask_r
                t = ky * 3 + kx
                wt = w_ref[pl.ds(si * 288 + t * _C, _C), :]
                contrib = lax.dot_general(hs, wt, (((1,), (0,)), ((), ())),
                                          precision=None)
                conv = contrib if conv is None else conv + contrib
        conv = conv + b_ref[pl.ds(si, 1), :]
        h2 = h * (1.0 - 0.5) + conv * 0.5

        diff = h2 - resid
        loss_acc = loss_acc + jnp.sum(diff * diff, axis=(0, 1), keepdims=True)
        recon = recon + h2
        resid = resid - h2

    y_ref[0] = x_img + (recon - x_img)
    loss_ref[0] = loss_acc


@jax.jit
def kernel(x, emb, phi_w, phi_b):
    x_rows = x.transpose(0, 2, 3, 1).reshape(_B, _NPIX, _C)
    embt = emb.T
    w_sel = phi_w[jnp.array(_KIDX)]
    w_cat = w_sel.transpose(0, 3, 4, 2, 1).reshape(_NSCALES * 288, _C)
    b_cat = phi_b[jnp.array(_KIDX)]

    u_ops = [jnp.asarray(u) for u in _U_MATS]
    p_ops = [jnp.asarray(p) for p in _P_MATS]

    const = lambda *shape: pl.BlockSpec(shape, lambda i: tuple(0 for _ in shape))

    in_specs = [
            pl.BlockSpec((1, _NPIX, _C), lambda i: (i, 0, 0)),
            const(_VOCAB, _C),
            const(_C, _VOCAB),
            const(_NSCALES * 288, _C),
            const(_NSCALES, _C),
    ] + [const(_NPIX, pn * pn) for pn in _PATCH[:-1]] \
      + [const(pn * pn, _NPIX) for pn in _PATCH[:-1]]
    out_specs = [
        pl.BlockSpec((1, _NPIX, _C), lambda i: (i, 0, 0)),
        pl.BlockSpec((1, 1, 1), lambda i: (i, 0, 0)),
    ]

    y_rows, loss_parts = pl.pallas_call(
        _vq_body,
        grid=(_B,),
        in_specs=in_specs,
        out_specs=out_specs,
        out_shape=[
            jax.ShapeDtypeStruct((_B, _NPIX, _C), jnp.float32),
            jax.ShapeDtypeStruct((_B, 1, 1), jnp.float32),
        ],
        scratch_shapes=[pltpu.VMEM((_NPIX + 80, _C), jnp.float32)],
        compiler_params=pltpu.CompilerParams(
            dimension_semantics=("arbitrary",),
            vmem_limit_bytes=100 * 1024 * 1024,
        ),
    )(x_rows, emb, embt, w_cat, b_cat, *u_ops, *p_ops)

    numel = _B * _C * _HW * _HW
    total = jnp.sum(loss_parts)
    loss = total * ((1.0 + _BETA) / numel) * (1.0 / _NSCALES)
    y = y_rows.reshape(_B, _HW, _HW, _C).transpose(0, 3, 1, 2)
    return y, loss


# speedup vs baseline: 1.5232x; 1.5232x over previous
import functools

import numpy as np
import jax
import jax.numpy as jnp
from jax import lax
from jax.experimental import pallas as pl
from jax.experimental.pallas import tpu as pltpu

_PATCH = (1, 2, 4, 8, 16, 32)
_VOCAB = 8192
_C = 32
_HW = 32
_NPIX = _HW * _HW
_B = 16
_KIDX = (0, 0, 1, 2, 3, 3)
_BETA = 0.25
_NSCALES = 6


def _cubic_kernel(x):
    x = np.abs(x)
    out = np.where(x <= 1.0, ((1.5 * x - 2.5) * x) * x + 1.0, 0.0)
    out = np.where((x > 1.0) & (x < 2.0), ((-0.5 * x + 2.5) * x - 4.0) * x + 2.0, out)
    return out


def _resize_mat(n, m):
    sample_f = (np.arange(m, dtype=np.float64) + 0.5) * (n / m) - 0.5
    x = sample_f[None, :] - np.arange(n, dtype=np.float64)[:, None]
    w = _cubic_kernel(x)
    w = w / w.sum(axis=0, keepdims=True)
    return w.T


def _pool_mat(pn):
    f = _HW // pn
    a = np.zeros((pn, _HW), dtype=np.float64)
    for p in range(pn):
        a[p, p * f:(p + 1) * f] = 1.0 / f
    return a


_U_MATS = []
_P_MATS = []
for _pn in _PATCH[:-1]:
    _k = _resize_mat(_pn, _HW)
    _U_MATS.append(np.kron(_k, _k).astype(np.float32))
    _a = _pool_mat(_pn)
    _P_MATS.append(np.kron(_a, _a).astype(np.float32))

_CHUNK = 256


def _vq_body(x_ref, emb_ref, embt_ref, w_ref, b_ref,
             u0, u1, u2, u3, u4, p0, p1, p2, p3, p4,
             y_ref, loss_ref, pad_ref):
    u_refs = (u0, u1, u2, u3, u4)
    p_refs = (p0, p1, p2, p3, p4)

    emb = emb_ref[...]
    embt = embt_ref[...]
    e2 = jnp.sum(emb * emb, axis=1)[None, :]
    emb_hi = emb.astype(jnp.bfloat16).astype(jnp.float32)
    emb_lo = emb - emb_hi

    x_img = x_ref[0]
    resid = x_img
    recon = jnp.zeros((_NPIX, _C), jnp.float32)
    loss_acc = jnp.zeros((1, 1), jnp.float32)

    pad_ref[...] = jnp.zeros(pad_ref.shape, jnp.float32)

    ww = lax.broadcasted_iota(jnp.int32, (_NPIX, 1), 0) & (_HW - 1)
    mask_l = (ww >= 1).astype(jnp.float32)
    mask_r = (ww <= _HW - 2).astype(jnp.float32)

    for si, pn in enumerate(_PATCH):
        n = pn * pn
        if si < _NSCALES - 1:
            z = lax.dot_general(p_refs[si][...], resid,
                                (((1,), (0,)), ((), ())),
                                precision=lax.Precision.HIGHEST)
        else:
            z = resid
        x2 = jnp.sum(z * z, axis=1, keepdims=True)

        ch = min(n, _CHUNK)
        h_chunks = []
        for c0 in range(0, n, ch):
            zc = lax.slice(z, (c0, 0), (c0 + ch, _C))
            x2c = lax.slice(x2, (c0, 0), (c0 + ch, 1))
            xe = lax.dot_general(zc, embt, (((1,), (0,)), ((), ())),
                                 precision=None)
            d = (x2c + e2) - 2.0 * xe
            iv = lax.broadcasted_iota(jnp.int32, (ch, _VOCAB), 1)
            if si < _NSCALES - 1:
                m = jnp.min(d, axis=1, keepdims=True)
                idx = jnp.min(jnp.where(d == m, iv, _VOCAB), axis=1, keepdims=True)
            else:
                accv = acci = None
                for k in range(8):
                    dk = lax.slice(d, (0, k * 1024), (ch, (k + 1) * 1024))
                    ivk = lax.slice(iv, (0, k * 1024), (ch, (k + 1) * 1024))
                    mk = jnp.min(dk, axis=1, keepdims=True)
                    ik = jnp.min(jnp.where(dk == mk, ivk, _VOCAB),
                                 axis=1, keepdims=True)
                    if k == 0:
                        accv, acci = mk, ik
                    else:
                        take = mk <= accv
                        accv = jnp.where(take, mk, accv)
                        acci = jnp.where(take, ik, acci)
                    if k % 4 == 3 and k != 7:
                        accv = accv.astype(jnp.bfloat16).astype(jnp.float32)
                idx = acci
            oh = (iv == idx).astype(jnp.float32)
            dn = (((1,), (0,)), ((), ()))
            h_chunks.append(lax.dot_general(oh, emb_hi, dn, precision=None)
                            + lax.dot_general(oh, emb_lo, dn, precision=None))
        h_small = h_chunks[0] if len(h_chunks) == 1 else jnp.concatenate(h_chunks, 0)

        if si < _NSCALES - 1:
            h = lax.dot_general(u_refs[si][...], h_small,
                                (((1,), (0,)), ((), ())),
                                precision=lax.Precision.HIGHEST)
        else:
            h = h_small

        pad_ref[pl.ds(40, _NPIX), :] = h
        conv = None
        for ky in range(3):
            for kx in range(3):
                s = (ky - 1) * _HW + (kx - 1)
                hs = pad_ref[pl.ds(40 + s, _NPIX), :]
                if kx == 0:
                    hs = hs * mask_l
                elif kx == 2:
                    hs = hs * mask_r
                t = ky * 3 + kx
                wt = w_ref[pl.ds(si * 288 + t * _C, _C), :]
                contrib = lax.dot_general(hs, wt, (((1,), (0,)), ((), ())),
                                          precision=None)
                conv = contrib if conv is None else conv + contrib
        conv = conv + b_ref[pl.ds(si, 1), :]
        h2 = h * (1.0 - 0.5) + conv * 0.5

        diff = h2 - resid
        loss_acc = loss_acc + jnp.sum(diff * diff, axis=(0, 1), keepdims=True)
        recon = recon + h2
        resid = resid - h2

    y_ref[0] = x_img + (recon - x_img)
    loss_ref[0] = loss_acc


@jax.jit
def kernel(x, emb, phi_w, phi_b):
    x_rows = x.transpose(0, 2, 3, 1).reshape(_B, _NPIX, _C)
    embt = emb.T
    w_sel = phi_w[jnp.array(_KIDX)]
    w_cat = w_sel.transpose(0, 3, 4, 2, 1).reshape(_NSCALES * 288, _C)
    b_cat = phi_b[jnp.array(_KIDX)]

    u_ops = [jnp.asarray(u) for u in _U_MATS]
    p_ops = [jnp.asarray(p) for p in _P_MATS]

    const = lambda *shape: pl.BlockSpec(shape, lambda i: tuple(0 for _ in shape))

    in_specs = [
            pl.BlockSpec((1, _NPIX, _C), lambda i: (i, 0, 0)),
            const(_VOCAB, _C),
            const(_C, _VOCAB),
            const(_NSCALES * 288, _C),
            const(_NSCALES, _C),
    ] + [const(_NPIX, pn * pn) for pn in _PATCH[:-1]] \
      + [const(pn * pn, _NPIX) for pn in _PATCH[:-1]]
    out_specs = [
        pl.BlockSpec((1, _NPIX, _C), lambda i: (i, 0, 0)),
        pl.BlockSpec((1, 1, 1), lambda i: (i, 0, 0)),
    ]

    y_rows, loss_parts = pl.pallas_call(
        _vq_body,
        grid=(_B,),
        in_specs=in_specs,
        out_specs=out_specs,
        out_shape=[
            jax.ShapeDtypeStruct((_B, _NPIX, _C), jnp.float32),
            jax.ShapeDtypeStruct((_B, 1, 1), jnp.float32),
        ],
        scratch_shapes=[pltpu.VMEM((_NPIX + 80, _C), jnp.float32)],
        compiler_params=pltpu.CompilerParams(
            dimension_semantics=("parallel",),
            vmem_limit_bytes=100 * 1024 * 1024,
        ),
    )(x_rows, emb, embt, w_cat, b_cat, *u_ops, *p_ops)

    numel = _B * _C * _HW * _HW
    total = jnp.sum(loss_parts)
    loss = total * ((1.0 + _BETA) / numel) * (1.0 / _NSCALES)
    y = y_rows.reshape(_B, _HW, _HW, _C).transpose(0, 3, 1, 2)
    return y, loss


# speedup vs baseline: 1.5785x; 1.0363x over previous
import functools

import numpy as np
import jax
import jax.numpy as jnp
from jax import lax
from jax.experimental import pallas as pl
from jax.experimental.pallas import tpu as pltpu

_PATCH = (1, 2, 4, 8, 16, 32)
_VOCAB = 8192
_C = 32
_HW = 32
_NPIX = _HW * _HW
_B = 16
_KIDX = (0, 0, 1, 2, 3, 3)
_BETA = 0.25
_NSCALES = 6


def _cubic_kernel(x):
    x = np.abs(x)
    out = np.where(x <= 1.0, ((1.5 * x - 2.5) * x) * x + 1.0, 0.0)
    out = np.where((x > 1.0) & (x < 2.0), ((-0.5 * x + 2.5) * x - 4.0) * x + 2.0, out)
    return out


def _resize_mat(n, m):
    sample_f = (np.arange(m, dtype=np.float64) + 0.5) * (n / m) - 0.5
    x = sample_f[None, :] - np.arange(n, dtype=np.float64)[:, None]
    w = _cubic_kernel(x)
    w = w / w.sum(axis=0, keepdims=True)
    return w.T


def _pool_mat(pn):
    f = _HW // pn
    a = np.zeros((pn, _HW), dtype=np.float64)
    for p in range(pn):
        a[p, p * f:(p + 1) * f] = 1.0 / f
    return a


_U_MATS = []
_P_MATS = []
for _pn in _PATCH[:-1]:
    _k = _resize_mat(_pn, _HW)
    _U_MATS.append(np.kron(_k, _k).astype(np.float32))
    _a = _pool_mat(_pn)
    _P_MATS.append(np.kron(_a, _a).astype(np.float32))

_CHUNK = 256


def _vq_body(x_ref, emb_ref, embt_ref, w_ref, b_ref,
             u0, u1, u2, u3, u4, p0, p1, p2, p3, p4,
             y_ref, loss_ref, pad_ref):
    u_refs = (u0, u1, u2, u3, u4)
    p_refs = (p0, p1, p2, p3, p4)

    emb = emb_ref[...]
    embt = embt_ref[...]
    e2 = jnp.sum(emb * emb, axis=1)[None, :]
    emb_hi = emb.astype(jnp.bfloat16).astype(jnp.float32)
    emb_lo = emb - emb_hi

    x_img = x_ref[0]
    resid = x_img
    recon = jnp.zeros((_NPIX, _C), jnp.float32)
    loss_acc = jnp.zeros((1, 1), jnp.float32)

    pad_ref[...] = jnp.zeros(pad_ref.shape, jnp.float32)

    ww = lax.broadcasted_iota(jnp.int32, (_NPIX, 1), 0) & (_HW - 1)
    mask_l = (ww >= 1).astype(jnp.float32)
    mask_r = (ww <= _HW - 2).astype(jnp.float32)

    for si, pn in enumerate(_PATCH):
        n = pn * pn
        if si < _NSCALES - 1:
            z = lax.dot_general(p_refs[si][...], resid,
                                (((1,), (0,)), ((), ())),
                                precision=lax.Precision.HIGHEST)
        else:
            z = resid
        x2 = jnp.sum(z * z, axis=1, keepdims=True)

        ch = min(n, _CHUNK)
        h_chunks = []
        for c0 in range(0, n, ch):
            zc = lax.slice(z, (c0, 0), (c0 + ch, _C))
            x2c = lax.slice(x2, (c0, 0), (c0 + ch, 1))
            xe = lax.dot_general(zc, embt, (((1,), (0,)), ((), ())),
                                 precision=None)
            d = (x2c + e2) - 2.0 * xe
            iv = lax.broadcasted_iota(jnp.int32, (ch, _VOCAB), 1)
            if si < _NSCALES - 1:
                idx = jnp.argmin(d, axis=1)[:, None]
            else:
                accv = acci = None
                for k in range(8):
                    dk = lax.slice(d, (0, k * 1024), (ch, (k + 1) * 1024))
                    mk = jnp.min(dk, axis=1, keepdims=True)
                    ik = jnp.argmin(dk, axis=1)[:, None] + (k * 1024)
                    if k == 0:
                        accv, acci = mk, ik
                    else:
                        take = mk <= accv
                        accv = jnp.where(take, mk, accv)
                        acci = jnp.where(take, ik, acci)
                    if k % 4 == 3 and k != 7:
                        accv = accv.astype(jnp.bfloat16).astype(jnp.float32)
                idx = acci
            oh = (iv == idx).astype(jnp.float32)
            dn = (((1,), (0,)), ((), ()))
            h_chunks.append(lax.dot_general(oh, emb_hi, dn, precision=None)
                            + lax.dot_general(oh, emb_lo, dn, precision=None))
        h_small = h_chunks[0] if len(h_chunks) == 1 else jnp.concatenate(h_chunks, 0)

        if si < _NSCALES - 1:
            h = lax.dot_general(u_refs[si][...], h_small,
                                (((1,), (0,)), ((), ())),
                                precision=lax.Precision.HIGHEST)
        else:
            h = h_small

        pad_ref[pl.ds(40, _NPIX), :] = h
        conv = None
        for ky in range(3):
            for kx in range(3):
                s = (ky - 1) * _HW + (kx - 1)
                hs = pad_ref[pl.ds(40 + s, _NPIX), :]
                if kx == 0:
                    hs = hs * mask_l
                elif kx == 2:
                    hs = hs * mask_r
                t = ky * 3 + kx
                wt = w_ref[pl.ds(si * 288 + t * _C, _C), :]
                contrib = lax.dot_general(hs, wt, (((1,), (0,)), ((), ())),
                                          precision=None)
                conv = contrib if conv is None else conv + contrib
        conv = conv + b_ref[pl.ds(si, 1), :]
        h2 = h * (1.0 - 0.5) + conv * 0.5

        diff = h2 - resid
        loss_acc = loss_acc + jnp.sum(diff * diff, axis=(0, 1), keepdims=True)
        recon = recon + h2
        resid = resid - h2

    y_ref[0] = x_img + (recon - x_img)
    loss_ref[0] = loss_acc


@jax.jit
def kernel(x, emb, phi_w, phi_b):
    x_rows = x.transpose(0, 2, 3, 1).reshape(_B, _NPIX, _C)
    embt = emb.T
    w_sel = phi_w[jnp.array(_KIDX)]
    w_cat = w_sel.transpose(0, 3, 4, 2, 1).reshape(_NSCALES * 288, _C)
    b_cat = phi_b[jnp.array(_KIDX)]

    u_ops = [jnp.asarray(u) for u in _U_MATS]
    p_ops = [jnp.asarray(p) for p in _P_MATS]

    const = lambda *shape: pl.BlockSpec(shape, lambda i: tuple(0 for _ in shape))

    in_specs = [
            pl.BlockSpec((1, _NPIX, _C), lambda i: (i, 0, 0)),
            const(_VOCAB, _C),
            const(_C, _VOCAB),
            const(_NSCALES * 288, _C),
            const(_NSCALES, _C),
    ] + [const(_NPIX, pn * pn) for pn in _PATCH[:-1]] \
      + [const(pn * pn, _NPIX) for pn in _PATCH[:-1]]
    out_specs = [
        pl.BlockSpec((1, _NPIX, _C), lambda i: (i, 0, 0)),
        pl.BlockSpec((1, 1, 1), lambda i: (i, 0, 0)),
    ]

    y_rows, loss_parts = pl.pallas_call(
        _vq_body,
        grid=(_B,),
        in_specs=in_specs,
        out_specs=out_specs,
        out_shape=[
            jax.ShapeDtypeStruct((_B, _NPIX, _C), jnp.float32),
            jax.ShapeDtypeStruct((_B, 1, 1), jnp.float32),
        ],
        scratch_shapes=[pltpu.VMEM((_NPIX + 80, _C), jnp.float32)],
        compiler_params=pltpu.CompilerParams(
            dimension_semantics=("parallel",),
            vmem_limit_bytes=100 * 1024 * 1024,
        ),
    )(x_rows, emb, embt, w_cat, b_cat, *u_ops, *p_ops)

    numel = _B * _C * _HW * _HW
    total = jnp.sum(loss_parts)
    loss = total * ((1.0 + _BETA) / numel) * (1.0 / _NSCALES)
    y = y_rows.reshape(_B, _HW, _HW, _C).transpose(0, 3, 1, 2)
    return y, loss
